# Initial kernel scaffold; baseline (speedup 1.0000x reference)
import jax
import jax.numpy as jnp
from jax.experimental import pallas as pl


def kernel(inputs):
    raise NotImplementedError("write your pallas kernel here")


# speedup vs baseline: 10.9810x; 10.9810x over previous
import functools

import jax
import jax.numpy as jnp
import numpy as np
from jax import lax
from jax.experimental import pallas as pl
from jax.experimental.pallas import tpu as pltpu
from jax.experimental.pallas import tpu_sc as plsc

N = 1_000_000
KSEL = 512
W = 16
CBASE = 62512
CLAST = N - 15 * CBASE
CBUF = CBASE + 16
HD = 256
MININT = np.int32(-(2 ** 31))


def _keys_of(v):
    u = lax.bitcast_convert_type(v, jnp.int32)
    m = lax.shift_right_arithmetic(u, 31)
    return lax.bitwise_xor(u, lax.bitwise_or(m, MININT))


def _digit(key, shift):
    d = lax.shift_right_logical(key, shift)
    if shift != 24:
        d = lax.bitwise_and(d, jnp.int32(0xFF))
    return d


def _extract(vec, lane, lane_iota):
    return jnp.max(jnp.where(lane_iota == lane, vec, MININT))


def _sc_select_builder():
    mesh = plsc.VectorSubcoreMesh(
        core_axis_name="c", subcore_axis_name="s", num_cores=1,
        num_subcores=W)

    @functools.partial(
        pl.kernel,
        out_type=jax.ShapeDtypeStruct((16,), jnp.float32),
        mesh=mesh,
        compiler_params=pltpu.CompilerParams(needs_layout_passes=False),
        scratch_types=[
            pltpu.VMEM((CBUF,), jnp.float32),
            pltpu.VMEM((HD * 16,), jnp.int32),
            pltpu.VMEM((256,), jnp.int32),
            pltpu.VMEM((256,), jnp.int32),
            pltpu.VMEM((256,), jnp.int32),
            pltpu.VMEM((16,), jnp.float32),
            pltpu.VMEM((16,), jnp.int32),
            pltpu.VMEM_SHARED((W, HD * 16), jnp.int32),
            pltpu.VMEM_SHARED((256,), jnp.int32),
        ],
    )
    def sc_select(x_hbm, out_hbm, buf, hist, accv, stage, cnts, outv,
                  cvec_v, sh_hist, sh_cnt):
        w = lax.axis_index("s")
        lane = lax.iota(jnp.int32, 16)
        ones = jnp.ones((16,), jnp.int32)
        zero16 = jnp.zeros((16,), jnp.int32)
        base = w * CBASE
        m0 = jnp.where(w == W - 1, jnp.int32(CLAST), jnp.int32(CBASE))

        @pl.when(w < W - 1)
        def _():
            pltpu.sync_copy(x_hbm.at[pl.ds(base, CBASE)],
                            buf.at[pl.ds(0, CBASE)])

        @pl.when(w == W - 1)
        def _():
            pltpu.sync_copy(x_hbm.at[pl.ds(base, CLAST)],
                            buf.at[pl.ds(0, CLAST)])

        m_cur = m0
        kk = jnp.int32(KSEL)
        digits = []
        for shift in (24, 16, 8, 0):
            def zbody(j, _, hist=hist):
                hist[pl.ds(j * 16, 16)] = zero16
                return 0
            lax.fori_loop(0, HD, zbody, 0)

            nv = lax.div(m_cur + 15, jnp.int32(16))

            def hbody(i, _, shift=shift):
                v = buf[pl.ds(i * 16, 16)]
                dig = _digit(_keys_of(v), shift)
                idx = dig * 16 + lane
                valid = (i * 16 + lane) < m_cur
                plsc.addupdate_scatter(hist, [idx], ones, mask=valid)
                return 0
            lax.fori_loop(0, nv, hbody, 0)

            pltpu.sync_copy(hist, sh_hist.at[w])
            plsc.subcore_barrier()

            def abody(j, _):
                accv[pl.ds(j * 16, 16)] = zero16
                return 0
            lax.fori_loop(0, 16, abody, 0)

            def rbody(v, _):
                pltpu.sync_copy(sh_hist.at[v, pl.ds(w * 256, 256)], stage)

                def addb(j, _):
                    accv[pl.ds(j * 16, 16)] = (accv[pl.ds(j * 16, 16)]
                                               + stage[pl.ds(j * 16, 16)])
                    return 0
                lax.fori_loop(0, 16, addb, 0)
                return 0
            lax.fori_loop(0, W, rbody, 0)

            def cbody(j, cvec):
                t = jnp.sum(accv[pl.ds(j * 16, 16)])
                return jnp.where(lane == j, t, cvec)
            cnt_vec = lax.fori_loop(0, 16, cbody, zero16)
            cvec_v[...] = cnt_vec
            pltpu.sync_copy(cvec_v, sh_cnt.at[pl.ds(w * 16, 16)])
            plsc.subcore_barrier()

            pltpu.sync_copy(sh_cnt, cnts)

            def sbody(j, svec):
                t = jnp.sum(cnts[pl.ds(j * 16, 16)])
                return jnp.where(lane == j, t, svec)
            svec = lax.fori_loop(0, 16, sbody, zero16)
            suf_g = lax.rev(plsc.cumsum(lax.rev(svec, (0,))), (0,))
            jsel = jnp.max(jnp.where(suf_g >= kk, lane, jnp.int32(-1)))
            above_g = (_extract(suf_g, jsel, lane)
                       - _extract(svec, jsel, lane))
            kk2 = kk - above_g
            cg = cnts[pl.ds(jsel * 16, 16)]
            suf_l = lax.rev(plsc.cumsum(lax.rev(cg, (0,))), (0,))
            lsel = jnp.max(jnp.where(suf_l >= kk2, lane, jnp.int32(-1)))
            above_l = (_extract(suf_l, lsel, lane)
                       - _extract(cg, lsel, lane))
            d_sel = jsel * 16 + lsel
            kk = kk2 - above_l
            digits.append(d_sel)

            if shift != 0:
                def pbody(i, off, shift=shift, d_sel=d_sel, m_prev=m_cur):
                    v = buf[pl.ds(i * 16, 16)]
                    dig = _digit(_keys_of(v), shift)
                    valid = (i * 16 + lane) < m_prev
                    match = jnp.logical_and(dig == d_sel, valid)
                    plsc.store_compressed(buf.at[pl.ds(off, 16)], v,
                                          mask=match)
                    pc = plsc.all_reduce_population_count(match)
                    return off + jnp.max(pc)
                m_cur = lax.fori_loop(0, nv, pbody, jnp.int32(0))

        d0, d1, d2, d3 = digits
        tk = (lax.shift_left(d0, 24) | lax.shift_left(d1, 16)
              | lax.shift_left(d2, 8) | d3)
        tkv = jnp.full((16,), tk, jnp.int32)
        bits = jnp.where(tkv < 0, lax.bitwise_xor(tkv, MININT),
                         lax.bitwise_not(tkv))
        outv[...] = lax.bitcast_convert_type(bits, jnp.float32)

        @pl.when(w == 0)
        def _():
            pltpu.sync_copy(outv, out_hbm)

    return sc_select


_sc_select_cache = []


def _sc_select(x):
    if not _sc_select_cache:
        _sc_select_cache.append(_sc_select_builder())
    return _sc_select_cache[0](x)


def _mask_body(t_ref, x_ref, o_ref):
    t = t_ref[0]
    x = x_ref[...]
    o_ref[...] = jnp.where(x >= t, x, jnp.float32(0.0))


def _mask(t1, x2d):
    return pl.pallas_call(
        _mask_body,
        grid=(5,),
        in_specs=[
            pl.BlockSpec(memory_space=pltpu.SMEM),
            pl.BlockSpec((200, 1000), lambda i: (i, 0)),
        ],
        out_specs=pl.BlockSpec((200, 1000), lambda i: (i, 0)),
        out_shape=jax.ShapeDtypeStruct((1000, 1000), jnp.float32),
    )(t1, x2d)


@jax.jit
def kernel(inputs):
    thr16 = _sc_select(inputs)
    t1 = lax.slice(thr16, (0,), (1,))
    out2d = _mask(t1, inputs.reshape(1000, 1000))
    return out2d.reshape(N)
